# speedup vs baseline: 2.9541x; 2.9541x over previous
import jax
import jax.numpy as jnp
from jax import lax
from jax.experimental import pallas as pl
from jax.experimental.pallas import tpu as pltpu
from jax.experimental.pallas import tpu_sc as plsc

NC = 2
NS = 16
NW = NC * NS


def _make_sc_broadcast(B, S, D, chunk):
    rows_per_w = S // NW
    n_chunks = rows_per_w // chunk
    mesh = plsc.VectorSubcoreMesh(core_axis_name="c", subcore_axis_name="s")

    def body(table_hbm, out_hbm, buf, wsem):
        wid = lax.axis_index("s") * NC + lax.axis_index("c")
        base = wid * rows_per_w
        for c in range(n_chunks):
            s0 = base + c * chunk
            pltpu.sync_copy(table_hbm.at[pl.ds(s0, chunk)], buf)
            copies = [
                pltpu.make_async_copy(buf, out_hbm.at[b, pl.ds(s0, chunk)], wsem)
                for b in range(B)
            ]
            for cp in copies:
                cp.start()
            for cp in copies:
                cp.wait()

    return pl.kernel(
        body,
        out_type=jax.ShapeDtypeStruct((B, S, D), jnp.float32),
        mesh=mesh,
        scratch_types=[
            pltpu.VMEM((chunk, D), jnp.float32),
            pltpu.SemaphoreType.DMA,
        ],
    )


def kernel(x, pos_table):
    B, S, D = x.shape
    return _make_sc_broadcast(B, S, D, chunk=64)(pos_table)


# speedup vs baseline: 2.9729x; 1.0064x over previous
import jax
import jax.numpy as jnp
from jax import lax
from jax.experimental import pallas as pl
from jax.experimental.pallas import tpu as pltpu
from jax.experimental.pallas import tpu_sc as plsc

NC = 2
NS = 16
NW = NC * NS


def _make_sc_broadcast(B, S, D, chunk, nbuf):
    rows_per_w = S // NW
    n_chunks = rows_per_w // chunk
    mesh = plsc.VectorSubcoreMesh(core_axis_name="c", subcore_axis_name="s")

    def body(table_hbm, out_hbm, *rest):
        bufs, (rsem, wsem) = rest[:nbuf], rest[nbuf:]
        wid = lax.axis_index("s") * NC + lax.axis_index("c")
        base = wid * rows_per_w

        def read(c):
            return pltpu.make_async_copy(
                table_hbm.at[pl.ds(base + c * chunk, chunk)], bufs[c % nbuf], rsem
            )

        def writes(c):
            return [
                pltpu.make_async_copy(
                    bufs[c % nbuf], out_hbm.at[b, pl.ds(base + c * chunk, chunk)], wsem
                )
                for b in range(B)
            ]

        for c in range(min(nbuf, n_chunks)):
            read(c).start()
        drained = 0
        for c in range(n_chunks):
            read(c).wait()
            nxt = c - 1 + nbuf
            if c >= 1 and nxt < n_chunks:
                for cp in writes(c - 1):
                    cp.wait()
                drained = c
                read(nxt).start()
            for cp in writes(c):
                cp.start()
        for c in range(drained, n_chunks):
            for cp in writes(c):
                cp.wait()

    return pl.kernel(
        body,
        out_type=jax.ShapeDtypeStruct((B, S, D), jnp.float32),
        mesh=mesh,
        scratch_types=[pltpu.VMEM((chunk, D), jnp.float32)] * nbuf
        + [pltpu.SemaphoreType.DMA, pltpu.SemaphoreType.DMA],
    )


def kernel(x, pos_table):
    B, S, D = x.shape
    return _make_sc_broadcast(B, S, D, chunk=32, nbuf=3)(pos_table)
